# baseline (speedup 1.0000x reference)
import functools

import jax
import jax.numpy as jnp
from jax import lax
from jax.experimental import pallas as pl
from jax.experimental.pallas import tpu as pltpu
from jax.experimental.pallas import tpu_sc as plsc

B, L, D, K, LP = 1024, 200, 128, 2, 64
H = D + K * LP
VOCAB = 100000
FEAT_ID, TGT_ID = 3, 4
EPS = 1e-12

N = B * L
CHUNK = 128


def _sc_gather(table, idx_flat):
  info = plsc.get_sparse_core_info()
  nw = info.num_cores * info.num_subcores
  n_chunks = N // CHUNK
  chunks_per_w = n_chunks // nw
  rows_per_w = chunks_per_w * CHUNK
  idx3d = idx_flat.reshape(nw, chunks_per_w, CHUNK)

  mesh = plsc.VectorSubcoreMesh(core_axis_name="c", subcore_axis_name="s")

  @functools.partial(
      pl.kernel,
      mesh=mesh,
      out_type=jax.ShapeDtypeStruct((N, D), jnp.float32),
      scratch_types=[
          pltpu.VMEM((chunks_per_w, CHUNK), jnp.int32),
          pltpu.VMEM((CHUNK, D), jnp.float32),
          pltpu.SemaphoreType.DMA,
      ],
  )
  def k(table_hbm, idx_hbm, out_hbm, idx_v, rows_v, sem):
    wid = lax.axis_index("s") * info.num_cores + lax.axis_index("c")
    base_row = wid * rows_per_w
    pltpu.sync_copy(idx_hbm.at[wid], idx_v)

    @pl.loop(0, chunks_per_w)
    def _(j):
      pltpu.async_copy(table_hbm.at[idx_v.at[j]], rows_v, sem).wait()
      pltpu.sync_copy(rows_v, out_hbm.at[pl.ds(base_row + j * CHUNK, CHUNK)])

  return k(table, idx3d)


def _dotT(a, b):
  return lax.dot_general(a, b, (((0,), (0,)), ((), ())),
                         preferred_element_type=jnp.float32)


def _tc_body(g_ref, w1_ref, w2_ref, lp_ref, mol_ref, tgt_ref, cat_ref,
             gam_ref, bet_ref, out_ref):
  bb = g_ref.shape[0]
  w1 = w1_ref[...]
  w2 = w2_ref[...]
  tt = w1 & 15
  scale = (1.0 + mol_ref[...] * (tt == FEAT_ID).astype(jnp.float32)
           + tgt_ref[...] * (tt == TGT_ID).astype(jnp.float32))
  p0 = w2 & 0xFFFF
  p1 = w2 >> 16
  cat = cat_ref[...].astype(jnp.bfloat16)
  gam = gam_ref[...]
  bet = bet_ref[...]
  iota16 = lax.broadcasted_iota(jnp.int32, (16, L), 0)
  iota_l = lax.broadcasted_iota(jnp.int32, (L, L), 0)
  ones_d = jnp.ones((1, D), jnp.bfloat16)

  for j in range(bb):
    w1j = w1[j:j + 1]
    mt = jnp.concatenate(
        [iota16 == ((w1j >> (4 * t)) & 15) for t in range(8)],
        axis=0).astype(jnp.bfloat16)
    small = _dotT(mt, cat)
    scale_mat = _dotT(scale[j:j + 1].astype(jnp.bfloat16), ones_d)
    dense = g_ref[j] * scale_mat + small

    lp = lp_ref[j].astype(jnp.bfloat16)
    q0 = _dotT((iota_l == p0[j:j + 1]).astype(jnp.bfloat16), lp)
    q1 = _dotT((iota_l == p1[j:j + 1]).astype(jnp.bfloat16), lp)

    emb = jnp.concatenate([dense, q0, q1], axis=-1)
    mean = jnp.mean(emb, axis=-1, keepdims=True)
    var = jnp.mean(emb * emb, axis=-1, keepdims=True) - mean * mean
    inv = lax.rsqrt(var + EPS)
    out_ref[j] = (emb - mean) * inv * gam + bet


def kernel(input_ids, token_type_ids, pos_embed_ids, lp_embeds, atom_props,
           bond_props, mol_features, target_values, emb_table, type_table,
           in_ring_table, charge_table, hybrid_table, chir_table,
           aromatic_table, conj_table, stereo_table, ln_gamma, ln_beta):
  g = _sc_gather(emb_table, input_ids.reshape(N).astype(jnp.int32)).reshape(B, L, D)

  cat = jnp.concatenate([
      jnp.pad(t, ((0, 16 - t.shape[0]), (0, 0)))
      for t in (type_table, in_ring_table, charge_table, hybrid_table,
                chir_table, aromatic_table, conj_table, stereo_table)],
      axis=0)
  w1 = (token_type_ids
        | (atom_props[..., 0] << 4) | (atom_props[..., 1] << 8)
        | (atom_props[..., 2] << 12) | (atom_props[..., 3] << 16)
        | (bond_props[..., 0] << 20) | (bond_props[..., 1] << 24)
        | (bond_props[..., 2] << 28)).astype(jnp.int32)
  w2 = (pos_embed_ids[..., 0] | (pos_embed_ids[..., 1] << 16)).astype(jnp.int32)
  gam2 = ln_gamma.reshape(1, H)
  bet2 = ln_beta.reshape(1, H)

  BB = 32
  grid = (B // BB,)
  out = pl.pallas_call(
      _tc_body,
      grid=grid,
      in_specs=[
          pl.BlockSpec((BB, L, D), lambda i: (i, 0, 0)),
          pl.BlockSpec((BB, L), lambda i: (i, 0)),
          pl.BlockSpec((BB, L), lambda i: (i, 0)),
          pl.BlockSpec((BB, L, LP), lambda i: (i, 0, 0)),
          pl.BlockSpec((BB, L), lambda i: (i, 0)),
          pl.BlockSpec((BB, L), lambda i: (i, 0)),
          pl.BlockSpec((128, D), lambda i: (0, 0)),
          pl.BlockSpec((1, H), lambda i: (0, 0)),
          pl.BlockSpec((1, H), lambda i: (0, 0)),
      ],
      out_specs=pl.BlockSpec((BB, L, H), lambda i: (i, 0, 0)),
      out_shape=jax.ShapeDtypeStruct((B, L, H), jnp.float32),
  )(g, w1, w2, lp_embeds, mol_features, target_values, cat, gam2, bet2)
  return out


# speedup vs baseline: 1.1001x; 1.1001x over previous
import functools

import jax
import jax.numpy as jnp
from jax import lax
from jax.experimental import pallas as pl
from jax.experimental.pallas import tpu as pltpu
from jax.experimental.pallas import tpu_sc as plsc

B, L, D, K, LP = 1024, 200, 128, 2, 64
H = D + K * LP
VOCAB = 100000
FEAT_ID, TGT_ID = 3, 4
EPS = 1e-12

N = B * L
CHUNK = 128
NBUF = 4


def _sc_gather(table, idx_flat, n_tokens):
  info = plsc.get_sparse_core_info()
  nw = info.num_cores * info.num_subcores
  chunks_per_w = n_tokens // (nw * CHUNK)
  rows_per_w = chunks_per_w * CHUNK
  idx3d = idx_flat.reshape(nw, chunks_per_w, CHUNK)

  mesh = plsc.VectorSubcoreMesh(core_axis_name="c", subcore_axis_name="s")

  @functools.partial(
      pl.kernel,
      mesh=mesh,
      out_type=jax.ShapeDtypeStruct((n_tokens, D), jnp.float32),
      scratch_types=[
          pltpu.VMEM((chunks_per_w, CHUNK), jnp.int32),
          pltpu.VMEM((NBUF, CHUNK, D), jnp.float32),
      ] + [pltpu.SemaphoreType.DMA] * (2 * NBUF),
  )
  def k(table_hbm, idx_hbm, out_hbm, idx_v, rows_v, *sems):
    gs, ss = sems[:NBUF], sems[NBUF:]
    wid = lax.axis_index("s") * info.num_cores + lax.axis_index("c")
    base_row = wid * rows_per_w
    pltpu.sync_copy(idx_hbm.at[wid], idx_v)

    def g_copy(j, b):
      return pltpu.make_async_copy(
          table_hbm.at[idx_v.at[j]], rows_v.at[b], gs[b])

    def s_copy(j, b):
      return pltpu.make_async_copy(
          rows_v.at[b], out_hbm.at[pl.ds(base_row + j * CHUNK, CHUNK)], ss[b])

    nch = chunks_per_w
    g_copy(0, 0).start()
    g_copy(1, 1).start()

    n_up = ((nch + NBUF - 1) // NBUF) * NBUF

    @pl.loop(0, n_up, step=NBUF)
    def _(j0):
      for b in range(NBUF):
        j = j0 + b

        @pl.when(j < nch)
        def _():
          g_copy(j, b).wait()
          s_copy(j, b).start()

          @pl.when(j + 2 < nch)
          def _():
            @pl.when(j >= 2)
            def _():
              s_copy(j - 2, (b + 2) % NBUF).wait()
            g_copy(j + 2, (b + 2) % NBUF).start()

    s_copy(nch - 1, (nch - 1) % NBUF).wait()
    s_copy(nch - 2, (nch - 2) % NBUF).wait()

  return k(table, idx3d)


def _dotT(a, b):
  return lax.dot_general(a, b, (((0,), (0,)), ((), ())),
                         preferred_element_type=jnp.float32)


def _tc_body(g_ref, w1_ref, w2_ref, lp_ref, mol_ref, tgt_ref, cat_ref,
             gam_ref, bet_ref, out_ref):
  bb = g_ref.shape[0]
  w1 = w1_ref[...]
  w2 = w2_ref[...]
  tt = w1 & 15
  scale = (1.0 + mol_ref[...] * (tt == FEAT_ID).astype(jnp.float32)
           + tgt_ref[...] * (tt == TGT_ID).astype(jnp.float32))
  p0 = w2 & 0xFFFF
  p1 = w2 >> 16
  cat = cat_ref[...].astype(jnp.bfloat16)
  gam = gam_ref[...]
  bet = bet_ref[...]
  iota16 = lax.broadcasted_iota(jnp.int32, (16, L), 0)
  iota_l = lax.broadcasted_iota(jnp.int32, (L, L), 0)
  ones_d = jnp.ones((1, D), jnp.bfloat16)

  for j in range(bb):
    w1j = w1[j:j + 1]
    mt = jnp.concatenate(
        [iota16 == ((w1j >> (4 * t)) & 15) for t in range(8)],
        axis=0).astype(jnp.bfloat16)
    small = _dotT(mt, cat)
    scale_mat = _dotT(scale[j:j + 1].astype(jnp.bfloat16), ones_d)
    dense = g_ref[j] * scale_mat + small

    lp = lp_ref[j].astype(jnp.bfloat16)
    q0 = _dotT((iota_l == p0[j:j + 1]).astype(jnp.bfloat16), lp)
    q1 = _dotT((iota_l == p1[j:j + 1]).astype(jnp.bfloat16), lp)

    emb = jnp.concatenate([dense, q0, q1], axis=-1)
    mean = jnp.mean(emb, axis=-1, keepdims=True)
    var = jnp.mean(emb * emb, axis=-1, keepdims=True) - mean * mean
    inv = lax.rsqrt(var + EPS)
    out_ref[j] = (emb - mean) * inv * gam + bet


def _tc_body_acc(acc_ref, *refs):
  del acc_ref
  _tc_body(*refs)


BB = 8
NCHUNKS = 2
BC = B // NCHUNKS
GC = BC // BB


def kernel(input_ids, token_type_ids, pos_embed_ids, lp_embeds, atom_props,
           bond_props, mol_features, target_values, emb_table, type_table,
           in_ring_table, charge_table, hybrid_table, chir_table,
           aromatic_table, conj_table, stereo_table, ln_gamma, ln_beta):
  ids_flat = input_ids.reshape(N).astype(jnp.int32)
  nc = BC * L
  g_parts = [_sc_gather(emb_table, ids_flat[c * nc:(c + 1) * nc], nc)
             .reshape(BC, L, D) for c in range(NCHUNKS)]

  cat = jnp.concatenate([
      jnp.pad(t, ((0, 16 - t.shape[0]), (0, 0)))
      for t in (type_table, in_ring_table, charge_table, hybrid_table,
                chir_table, aromatic_table, conj_table, stereo_table)],
      axis=0)
  w1 = (token_type_ids
        | (atom_props[..., 0] << 4) | (atom_props[..., 1] << 8)
        | (atom_props[..., 2] << 12) | (atom_props[..., 3] << 16)
        | (bond_props[..., 0] << 20) | (bond_props[..., 1] << 24)
        | (bond_props[..., 2] << 28)).astype(jnp.int32)
  w2 = (pos_embed_ids[..., 0] | (pos_embed_ids[..., 1] << 16)).astype(jnp.int32)
  gam2 = ln_gamma.reshape(1, H)
  bet2 = ln_beta.reshape(1, H)

  def specs(c):
    off = c * GC
    return [
        pl.BlockSpec((BB, L, D), lambda i: (i, 0, 0)),
        pl.BlockSpec((BB, L), lambda i: (off + i, 0)),
        pl.BlockSpec((BB, L), lambda i: (off + i, 0)),
        pl.BlockSpec((BB, L, LP), lambda i: (off + i, 0, 0)),
        pl.BlockSpec((BB, L), lambda i: (off + i, 0)),
        pl.BlockSpec((BB, L), lambda i: (off + i, 0)),
        pl.BlockSpec((128, D), lambda i: (0, 0)),
        pl.BlockSpec((1, H), lambda i: (0, 0)),
        pl.BlockSpec((1, H), lambda i: (0, 0)),
    ]

  out_shape = jax.ShapeDtypeStruct((B, L, H), jnp.float32)
  acc = pl.pallas_call(
      _tc_body,
      grid=(GC,),
      in_specs=specs(0),
      out_specs=pl.BlockSpec((BB, L, H), lambda i: (i, 0, 0)),
      out_shape=out_shape,
  )(g_parts[0], w1, w2, lp_embeds, mol_features, target_values,
    cat, gam2, bet2)
  for c in range(1, NCHUNKS):
    off = c * GC
    acc = pl.pallas_call(
        _tc_body_acc,
        grid=(GC,),
        in_specs=[pl.BlockSpec(memory_space=pl.ANY)] + specs(c),
        out_specs=pl.BlockSpec((BB, L, H),
                               lambda i, off=off: (off + i, 0, 0)),
        out_shape=out_shape,
        input_output_aliases={0: 0},
    )(acc, g_parts[c], w1, w2, lp_embeds, mol_features, target_values,
      cat, gam2, bet2)
  return acc


# speedup vs baseline: 1.1137x; 1.0124x over previous
import functools

import jax
import jax.numpy as jnp
from jax import lax
from jax.experimental import pallas as pl
from jax.experimental.pallas import tpu as pltpu
from jax.experimental.pallas import tpu_sc as plsc

B, L, D, K, LP = 1024, 200, 128, 2, 64
H = D + K * LP
VOCAB = 100000
FEAT_ID, TGT_ID = 3, 4
EPS = 1e-12

N = B * L
CHUNK = 128
NBUF = 4


def _sc_gather(table, idx_flat, n_tokens):
  info = plsc.get_sparse_core_info()
  nw = info.num_cores * info.num_subcores
  chunks_per_w = n_tokens // (nw * CHUNK)
  rows_per_w = chunks_per_w * CHUNK
  idx3d = idx_flat.reshape(nw, chunks_per_w, CHUNK)

  mesh = plsc.VectorSubcoreMesh(core_axis_name="c", subcore_axis_name="s")

  @functools.partial(
      pl.kernel,
      mesh=mesh,
      out_type=jax.ShapeDtypeStruct((n_tokens, D), jnp.float32),
      scratch_types=[
          pltpu.VMEM((chunks_per_w, CHUNK), jnp.int32),
          pltpu.VMEM((NBUF, CHUNK, D), jnp.float32),
      ] + [pltpu.SemaphoreType.DMA] * (2 * NBUF),
  )
  def k(table_hbm, idx_hbm, out_hbm, idx_v, rows_v, *sems):
    gs, ss = sems[:NBUF], sems[NBUF:]
    wid = lax.axis_index("s") * info.num_cores + lax.axis_index("c")
    base_row = wid * rows_per_w
    pltpu.sync_copy(idx_hbm.at[wid], idx_v)

    def g_copy(j, b):
      return pltpu.make_async_copy(
          table_hbm.at[idx_v.at[j]], rows_v.at[b], gs[b])

    def s_copy(j, b):
      return pltpu.make_async_copy(
          rows_v.at[b], out_hbm.at[pl.ds(base_row + j * CHUNK, CHUNK)], ss[b])

    nch = chunks_per_w
    g_copy(0, 0).start()
    g_copy(1, 1).start()

    n_up = ((nch + NBUF - 1) // NBUF) * NBUF

    @pl.loop(0, n_up, step=NBUF)
    def _(j0):
      for b in range(NBUF):
        j = j0 + b

        @pl.when(j < nch)
        def _():
          g_copy(j, b).wait()
          s_copy(j, b).start()

          @pl.when(j + 2 < nch)
          def _():
            @pl.when(j >= 2)
            def _():
              s_copy(j - 2, (b + 2) % NBUF).wait()
            g_copy(j + 2, (b + 2) % NBUF).start()

    s_copy(nch - 1, (nch - 1) % NBUF).wait()
    s_copy(nch - 2, (nch - 2) % NBUF).wait()

  return k(table, idx3d)


def _dotT(a, b):
  return lax.dot_general(a, b, (((0,), (0,)), ((), ())),
                         preferred_element_type=jnp.float32)


def _tc_body(g_ref, w1_ref, w2_ref, lp_ref, mol_ref, tgt_ref, cat_ref,
             gam_ref, bet_ref, out_ref):
  bb = g_ref.shape[0]
  w1 = w1_ref[...]
  w2 = w2_ref[...]
  tt = w1 & 15
  scale = (1.0 + mol_ref[...] * (tt == FEAT_ID).astype(jnp.float32)
           + tgt_ref[...] * (tt == TGT_ID).astype(jnp.float32))
  p0 = w2 & 0xFFFF
  p1 = w2 >> 16
  cat = cat_ref[...].astype(jnp.bfloat16)
  gam = gam_ref[...]
  bet = bet_ref[...]
  iota16 = lax.broadcasted_iota(jnp.int32, (16, L), 0)
  iota_l = lax.broadcasted_iota(jnp.int32, (L, L), 0)
  ones_d = jnp.ones((1, D), jnp.bfloat16)

  for j in range(bb):
    w1j = w1[j:j + 1]
    mt = jnp.concatenate(
        [iota16 == ((w1j >> (4 * t)) & 15) for t in range(8)],
        axis=0).astype(jnp.bfloat16)
    small = _dotT(mt, cat)
    scale_mat = _dotT(scale[j:j + 1].astype(jnp.bfloat16), ones_d)
    dense = g_ref[j] * scale_mat + small

    lp = lp_ref[j].astype(jnp.bfloat16)
    q0 = _dotT((iota_l == p0[j:j + 1]).astype(jnp.bfloat16), lp)
    q1 = _dotT((iota_l == p1[j:j + 1]).astype(jnp.bfloat16), lp)

    emb = jnp.concatenate([dense, q0, q1], axis=-1)
    mean = jnp.mean(emb, axis=-1, keepdims=True)
    var = jnp.mean(emb * emb, axis=-1, keepdims=True) - mean * mean
    inv = lax.rsqrt(var + EPS)
    out_ref[j] = (emb - mean) * inv * gam + bet


def _tc_body_acc(acc_ref, *refs):
  del acc_ref
  _tc_body(*refs)


BB = 8
NCHUNKS = 1
BC = B // NCHUNKS
GC = BC // BB


def kernel(input_ids, token_type_ids, pos_embed_ids, lp_embeds, atom_props,
           bond_props, mol_features, target_values, emb_table, type_table,
           in_ring_table, charge_table, hybrid_table, chir_table,
           aromatic_table, conj_table, stereo_table, ln_gamma, ln_beta):
  ids_flat = input_ids.reshape(N).astype(jnp.int32)
  nc = BC * L
  g_parts = [_sc_gather(emb_table, ids_flat[c * nc:(c + 1) * nc], nc)
             .reshape(BC, L, D) for c in range(NCHUNKS)]

  cat = jnp.concatenate([
      jnp.pad(t, ((0, 16 - t.shape[0]), (0, 0)))
      for t in (type_table, in_ring_table, charge_table, hybrid_table,
                chir_table, aromatic_table, conj_table, stereo_table)],
      axis=0)
  w1 = (token_type_ids
        | (atom_props[..., 0] << 4) | (atom_props[..., 1] << 8)
        | (atom_props[..., 2] << 12) | (atom_props[..., 3] << 16)
        | (bond_props[..., 0] << 20) | (bond_props[..., 1] << 24)
        | (bond_props[..., 2] << 28)).astype(jnp.int32)
  w2 = (pos_embed_ids[..., 0] | (pos_embed_ids[..., 1] << 16)).astype(jnp.int32)
  gam2 = ln_gamma.reshape(1, H)
  bet2 = ln_beta.reshape(1, H)

  def specs(c):
    off = c * GC
    return [
        pl.BlockSpec((BB, L, D), lambda i: (i, 0, 0)),
        pl.BlockSpec((BB, L), lambda i: (off + i, 0)),
        pl.BlockSpec((BB, L), lambda i: (off + i, 0)),
        pl.BlockSpec((BB, L, LP), lambda i: (off + i, 0, 0)),
        pl.BlockSpec((BB, L), lambda i: (off + i, 0)),
        pl.BlockSpec((BB, L), lambda i: (off + i, 0)),
        pl.BlockSpec((128, D), lambda i: (0, 0)),
        pl.BlockSpec((1, H), lambda i: (0, 0)),
        pl.BlockSpec((1, H), lambda i: (0, 0)),
    ]

  out_shape = jax.ShapeDtypeStruct((B, L, H), jnp.float32)
  acc = pl.pallas_call(
      _tc_body,
      grid=(GC,),
      in_specs=specs(0),
      out_specs=pl.BlockSpec((BB, L, H), lambda i: (i, 0, 0)),
      out_shape=out_shape,
  )(g_parts[0], w1, w2, lp_embeds, mol_features, target_values,
    cat, gam2, bet2)
  for c in range(1, NCHUNKS):
    off = c * GC
    acc = pl.pallas_call(
        _tc_body_acc,
        grid=(GC,),
        in_specs=[pl.BlockSpec(memory_space=pl.ANY)] + specs(c),
        out_specs=pl.BlockSpec((BB, L, H),
                               lambda i, off=off: (off + i, 0, 0)),
        out_shape=out_shape,
        input_output_aliases={0: 0},
    )(acc, g_parts[c], w1, w2, lp_embeds, mol_features, target_values,
      cat, gam2, bet2)
  return acc
